# baseline (speedup 1.0000x reference)
import functools

import jax
import jax.numpy as jnp
from jax import lax
from jax.experimental import pallas as pl
from jax.experimental.pallas import tpu as pltpu
from jax.experimental.pallas import tpu_sc as plsc

NC = 2
NS = 16
B = 128

N_PAD = 10240
ROWS_PER_TILE = N_PAD // NS

def _sc_mesh():
    return plsc.VectorSubcoreMesh(core_axis_name="c", subcore_axis_name="s",
                                  num_cores=NC, num_subcores=NS)


def _deg_body(dst4, ones_h, z_h, degp, idx_v, ones_v, deg_sh):
    c = lax.axis_index("c")
    s = lax.axis_index("s")
    nb = dst4.shape[2]
    sl = pl.ds(s * ROWS_PER_TILE, ROWS_PER_TILE)
    pltpu.sync_copy(z_h, deg_sh.at[sl])
    pltpu.sync_copy(ones_h, ones_v)
    pltpu.sync_copy(dst4.at[c, s], idx_v)
    plsc.subcore_barrier()

    def body(b, carry):
        pltpu.sync_copy(ones_v, deg_sh.at[idx_v.at[b]], add=True)
        return carry

    lax.fori_loop(0, nb, body, 0)
    plsc.subcore_barrier()
    pltpu.sync_copy(deg_sh.at[sl], degp.at[c, sl])


def _degree_kernel(dst4):
    nb = dst4.shape[2]
    k = pl.kernel(
        _deg_body,
        out_type=jax.ShapeDtypeStruct((NC, N_PAD, B), jnp.float32),
        mesh=_sc_mesh(),
        scratch_types=[
            pltpu.VMEM((nb, B), jnp.int32),
            pltpu.VMEM((B, B), jnp.float32),
            pltpu.VMEM_SHARED((N_PAD, B), jnp.float32),
        ],
    )
    ones_h = jnp.ones((B, B), jnp.float32)
    z_h = jnp.zeros((ROWS_PER_TILE, B), jnp.float32)
    return k(dst4, ones_h, z_h)


SB = 256


def _scatter_loop(ytab, src3, dst3, s, s_sh, sc_a, dc_a, sc_b, dc_b,
                  gbuf, gsem, isem_a, isem_b):
    nb = src3.shape[1]

    def stage(b, sc, dc, isem):
        pltpu.async_copy(src3.at[s, b], sc, isem)
        pltpu.async_copy(dst3.at[s, b], dc, isem)

    def stage_wait(b, sc, dc, isem):
        pltpu.make_async_copy(src3.at[s, b], sc, isem).wait()
        pltpu.make_async_copy(dst3.at[s, b], dc, isem).wait()

    stage(0, sc_a, dc_a, isem_a)

    def body(t, carry):
        b0 = 2 * t
        stage(b0 + 1, sc_b, dc_b, isem_b)
        stage_wait(b0, sc_a, dc_a, isem_a)
        pltpu.async_copy(ytab.at[sc_a], gbuf, gsem).wait()
        pltpu.sync_copy(gbuf, s_sh.at[dc_a], add=True)

        @pl.when(b0 + 2 < nb)
        def _():
            stage(b0 + 2, sc_a, dc_a, isem_a)

        stage_wait(b0 + 1, sc_b, dc_b, isem_b)
        pltpu.async_copy(ytab.at[sc_b], gbuf, gsem).wait()
        pltpu.sync_copy(gbuf, s_sh.at[dc_b], add=True)
        return carry

    lax.fori_loop(0, nb // 2, body, 0)


def _make_agg_body(n_chunks):
    nc2 = n_chunks // 2

    def body(y_all, src3, dst3, z_h, s_out, sc_a, dc_a, sc_b, dc_b,
             gbuf, s_sh, gsem, isem_a, isem_b):
        c = lax.axis_index("c")
        s = lax.axis_index("s")
        sl = pl.ds(s * ROWS_PER_TILE, ROWS_PER_TILE)
        for k in range(nc2):
            pltpu.sync_copy(z_h, s_sh.at[sl])
            plsc.subcore_barrier()

            @pl.when(c == 0)
            def _():
                _scatter_loop(y_all.at[k], src3, dst3, s, s_sh, sc_a, dc_a,
                              sc_b, dc_b, gbuf, gsem, isem_a, isem_b)

            @pl.when(c == 1)
            def _():
                _scatter_loop(y_all.at[nc2 + k], src3, dst3, s, s_sh, sc_a,
                              dc_a, sc_b, dc_b, gbuf, gsem, isem_a, isem_b)

            plsc.subcore_barrier()

            @pl.when(c == 0)
            def _():
                pltpu.sync_copy(s_sh.at[sl], s_out.at[k, sl])

            @pl.when(c == 1)
            def _():
                pltpu.sync_copy(s_sh.at[sl], s_out.at[nc2 + k, sl])

            if k + 1 < nc2:
                plsc.subcore_barrier()

    return body


def _aggregate(y_all, src3, dst3):
    n_chunks = y_all.shape[0]
    k = pl.kernel(
        _make_agg_body(n_chunks),
        out_type=jax.ShapeDtypeStruct((n_chunks, N_PAD, B), jnp.float32),
        mesh=_sc_mesh(),
        scratch_types=[
            pltpu.VMEM((SB,), jnp.int32),
            pltpu.VMEM((SB,), jnp.int32),
            pltpu.VMEM((SB,), jnp.int32),
            pltpu.VMEM((SB,), jnp.int32),
            pltpu.VMEM((SB, B), jnp.float32),
            pltpu.VMEM_SHARED((N_PAD, B), jnp.float32),
            pltpu.SemaphoreType.DMA,
            pltpu.SemaphoreType.DMA,
            pltpu.SemaphoreType.DMA,
        ],
    )
    z_h = jnp.zeros((ROWS_PER_TILE, B), jnp.float32)
    return k(y_all, src3, dst3, z_h)


def _dinv_block(degp_ref):
    deg = degp_ref[0, :, 0:1] + degp_ref[1, :, 0:1] + 1.0
    return lax.rsqrt(deg)


def _mm1_body(x_ref, w_ref, degp_ref, y_ref):
    acc = jnp.dot(x_ref[...], w_ref[...], preferred_element_type=jnp.float32)
    y_ref[0] = acc * _dinv_block(degp_ref)


def _layer1_matmul(xp, w1, degp, n_chunks):
    grid = (N_PAD // 256, n_chunks)
    return pl.pallas_call(
        _mm1_body,
        grid=grid,
        in_specs=[
            pl.BlockSpec((256, xp.shape[1]), lambda i, j: (i, 0)),
            pl.BlockSpec((xp.shape[1], B), lambda i, j: (0, j)),
            pl.BlockSpec((NC, 256, B), lambda i, j: (0, i, 0)),
        ],
        out_specs=pl.BlockSpec((1, 256, B), lambda i, j: (j, i, 0)),
        out_shape=jax.ShapeDtypeStruct((n_chunks, N_PAD, B), jnp.float32),
    )(xp, w1, degp)


def _mid_body(s1_ref, y1_ref, degp_ref, b1_ref, w2_ref, h_ref, y2_ref):
    dinv = _dinv_block(degp_ref)
    acc = jnp.zeros((s1_ref.shape[1], B), jnp.float32)
    for cch in range(s1_ref.shape[0]):
        hc = jax.nn.relu(dinv * (s1_ref[cch] + y1_ref[cch])
                         + b1_ref[0, cch * B:(cch + 1) * B])
        h_ref[:, cch * B:(cch + 1) * B] = hc
        acc = acc + jnp.dot(hc, w2_ref[cch],
                            preferred_element_type=jnp.float32)
    y2_ref[0] = acc * dinv


def _layer_mid(s1, y1c, degp, b1, w2r, n_out_chunks, n):
    n_chunks = s1.shape[0]
    hid = n_chunks * B
    R = 200
    grid = (n // R, n_out_chunks)
    return pl.pallas_call(
        _mid_body,
        grid=grid,
        in_specs=[
            pl.BlockSpec((n_chunks, R, B), lambda i, j: (0, i, 0)),
            pl.BlockSpec((n_chunks, R, B), lambda i, j: (0, i, 0)),
            pl.BlockSpec((NC, R, B), lambda i, j: (0, i, 0)),
            pl.BlockSpec((1, hid), lambda i, j: (0, 0)),
            pl.BlockSpec((n_chunks, B, B), lambda i, j: (0, 0, j)),
        ],
        out_specs=[
            pl.BlockSpec((R, hid), lambda i, j: (i, 0)),
            pl.BlockSpec((1, R, B), lambda i, j: (j, i, 0)),
        ],
        out_shape=[
            jax.ShapeDtypeStruct((n, hid), jnp.float32),
            jax.ShapeDtypeStruct((n_out_chunks, N_PAD, B), jnp.float32),
        ],
    )(s1, y1c, degp, b1, w2r)


def _final_body(s2_ref, y2_ref, degp_ref, b2_ref, out_ref):
    dinv = _dinv_block(degp_ref)
    for cch in range(s2_ref.shape[0]):
        out_ref[:, cch * B:(cch + 1) * B] = (
            dinv * (s2_ref[cch] + y2_ref[cch])
            + b2_ref[0, cch * B:(cch + 1) * B])


def _layer_final(s2, y2c, degp, b2, n):
    n_chunks = s2.shape[0]
    odim = n_chunks * B
    R = 200
    return pl.pallas_call(
        _final_body,
        grid=(n // R,),
        in_specs=[
            pl.BlockSpec((n_chunks, R, B), lambda i: (0, i, 0)),
            pl.BlockSpec((n_chunks, R, B), lambda i: (0, i, 0)),
            pl.BlockSpec((NC, R, B), lambda i: (0, i, 0)),
            pl.BlockSpec((1, odim), lambda i: (0, 0)),
        ],
        out_specs=pl.BlockSpec((R, odim), lambda i: (i, 0)),
        out_shape=jax.ShapeDtypeStruct((n, odim), jnp.float32),
    )(s2, y2c, degp, b2)


def kernel(x, edge_index, W1, b1, W2, b2):
    n, in_dim = x.shape
    hid = W1.shape[1]
    odim = W2.shape[1]
    e = edge_index.shape[1]

    epb_tile = -(-e // (NS * B))
    epb_tile += (-epb_tile) % 4
    e_pad = NS * epb_tile * B
    pad = jnp.full((e_pad - e,), n, jnp.int32)
    src = jnp.concatenate([edge_index[0].astype(jnp.int32), pad])
    dst = jnp.concatenate([edge_index[1].astype(jnp.int32), pad])
    src3 = src.reshape(NS, -1, SB)
    dst3 = dst.reshape(NS, -1, SB)
    dst4 = dst.reshape(NC, NS, epb_tile // 2, B)

    xp = jnp.zeros((N_PAD, in_dim), x.dtype).at[:n].set(x)
    b1r = b1.reshape(1, hid)
    b2r = b2.reshape(1, odim)
    w2r = W2.reshape(-1, B, odim)

    degp = _degree_kernel(dst4)
    n_chunks1 = hid // B
    y1c = _layer1_matmul(xp, W1, degp, n_chunks1)
    s1 = _aggregate(y1c, src3, dst3)
    n_chunks2 = odim // B
    h, y2c = _layer_mid(s1, y1c, degp, b1r, w2r, n_chunks2, n)
    s2 = _aggregate(y2c, src3, dst3)
    out = _layer_final(s2, y2c, degp, b2r, n)

    return (out, h)


# speedup vs baseline: 1.1082x; 1.1082x over previous
import functools

import jax
import jax.numpy as jnp
from jax import lax
from jax.experimental import pallas as pl
from jax.experimental.pallas import tpu as pltpu
from jax.experimental.pallas import tpu_sc as plsc

NC = 2
NS = 16
B = 128

N_PAD = 10240
ROWS_PER_TILE = N_PAD // NS

def _sc_mesh():
    return plsc.VectorSubcoreMesh(core_axis_name="c", subcore_axis_name="s",
                                  num_cores=NC, num_subcores=NS)


def _deg_body(dst4, ones_h, z_h, degp, idx_v, ones_v, deg_sh):
    c = lax.axis_index("c")
    s = lax.axis_index("s")
    nb = dst4.shape[2]
    pltpu.sync_copy(z_h, deg_sh.at[pl.ds(s * ROWS_PER_TILE, ROWS_PER_TILE)])
    pltpu.sync_copy(ones_h, ones_v)
    pltpu.sync_copy(dst4.at[c, s], idx_v)
    plsc.subcore_barrier()

    def body(b, carry):
        pltpu.sync_copy(ones_v, deg_sh.at[idx_v.at[b]], add=True)
        return carry

    lax.fori_loop(0, nb, body, 0)
    plsc.subcore_barrier()
    sl = pl.ds(s * ROWS_PER_TILE, ROWS_PER_TILE)
    pltpu.sync_copy(deg_sh.at[sl], degp.at[c, sl])


def _degree_kernel(dst4):
    nb = dst4.shape[2]
    k = pl.kernel(
        _deg_body,
        out_type=jax.ShapeDtypeStruct((NC, N_PAD, B), jnp.float32),
        mesh=_sc_mesh(),
        scratch_types=[
            pltpu.VMEM((nb, B), jnp.int32),
            pltpu.VMEM((B, B), jnp.float32),
            pltpu.VMEM_SHARED((N_PAD, B), jnp.float32),
        ],
    )
    ones_h = jnp.ones((B, B), jnp.float32)
    z_h = jnp.zeros((ROWS_PER_TILE, B), jnp.float32)
    return k(dst4, ones_h, z_h)


G = 16


def _scatter_loop(ytab, src3, dst3, s, s_sh, src_v, dst_v,
                  gbuf_a, gbuf_b, sem_a, sem_b):
    nb = src3.shape[1]

    def group(g, carry):
        pltpu.sync_copy(src3.at[s, pl.ds(g * G, G)], src_v)
        pltpu.sync_copy(dst3.at[s, pl.ds(g * G, G)], dst_v)
        pltpu.async_copy(ytab.at[src_v.at[0]], gbuf_a, sem_a)

        def body(t, carry2):
            b0 = 2 * t
            pltpu.async_copy(ytab.at[src_v.at[b0 + 1]], gbuf_b, sem_b)
            pltpu.make_async_copy(ytab.at[src_v.at[b0]], gbuf_a, sem_a).wait()
            pltpu.sync_copy(gbuf_a, s_sh.at[dst_v.at[b0]], add=True)

            @pl.when(b0 + 2 < G)
            def _():
                pltpu.async_copy(ytab.at[src_v.at[b0 + 2]], gbuf_a, sem_a)

            pltpu.make_async_copy(ytab.at[src_v.at[b0 + 1]], gbuf_b, sem_b).wait()
            pltpu.sync_copy(gbuf_b, s_sh.at[dst_v.at[b0 + 1]], add=True)
            return carry2

        lax.fori_loop(0, G // 2, body, 0)
        return carry

    lax.fori_loop(0, nb // G, group, 0)


def _make_agg_body(n_chunks):
    nc2 = n_chunks // 2

    def body(*refs):
        ytabs = refs[:n_chunks]
        src3, dst3, z_h, s_out = refs[n_chunks:n_chunks + 4]
        src_v, dst_v, gbuf_a, gbuf_b, s_sh, sem_a, sem_b = refs[n_chunks + 4:]
        c = lax.axis_index("c")
        s = lax.axis_index("s")
        sl = pl.ds(s * ROWS_PER_TILE, ROWS_PER_TILE)
        for k in range(nc2):
            pltpu.sync_copy(z_h, s_sh.at[sl])
            plsc.subcore_barrier()

            @pl.when(c == 0)
            def _():
                _scatter_loop(ytabs[k], src3, dst3, s, s_sh, src_v, dst_v,
                              gbuf_a, gbuf_b, sem_a, sem_b)

            @pl.when(c == 1)
            def _():
                _scatter_loop(ytabs[nc2 + k], src3, dst3, s, s_sh, src_v,
                              dst_v, gbuf_a, gbuf_b, sem_a, sem_b)

            plsc.subcore_barrier()

            @pl.when(c == 0)
            def _():
                pltpu.sync_copy(s_sh.at[sl], s_out.at[k, sl])

            @pl.when(c == 1)
            def _():
                pltpu.sync_copy(s_sh.at[sl], s_out.at[nc2 + k, sl])

            if k + 1 < nc2:
                plsc.subcore_barrier()

    return body


def _aggregate(ychunks, src3, dst3):
    n_chunks = len(ychunks)
    nb = src3.shape[1]
    k = pl.kernel(
        _make_agg_body(n_chunks),
        out_type=jax.ShapeDtypeStruct((n_chunks, N_PAD, B), jnp.float32),
        mesh=_sc_mesh(),
        scratch_types=[
            pltpu.VMEM((G, B), jnp.int32),
            pltpu.VMEM((G, B), jnp.int32),
            pltpu.VMEM((B, B), jnp.float32),
            pltpu.VMEM((B, B), jnp.float32),
            pltpu.VMEM_SHARED((N_PAD, B), jnp.float32),
            pltpu.SemaphoreType.DMA,
            pltpu.SemaphoreType.DMA,
        ],
    )
    z_h = jnp.zeros((ROWS_PER_TILE, B), jnp.float32)
    return k(*ychunks, src3, dst3, z_h)


def _dinv_block(degp_ref):
    deg = degp_ref[0, :, 0:1] + degp_ref[1, :, 0:1] + 1.0
    return lax.rsqrt(deg)


def _mm1_body(x_ref, w_ref, degp_ref, y_ref):
    acc = jnp.dot(x_ref[...], w_ref[...], preferred_element_type=jnp.float32)
    y_ref[0] = acc * _dinv_block(degp_ref)


def _layer1_matmul(xp, w1, degp, n_chunks):
    grid = (N_PAD // 256, n_chunks)
    return pl.pallas_call(
        _mm1_body,
        grid=grid,
        in_specs=[
            pl.BlockSpec((256, xp.shape[1]), lambda i, j: (i, 0)),
            pl.BlockSpec((xp.shape[1], B), lambda i, j: (0, j)),
            pl.BlockSpec((NC, 256, B), lambda i, j: (0, i, 0)),
        ],
        out_specs=pl.BlockSpec((1, 256, B), lambda i, j: (j, i, 0)),
        out_shape=jax.ShapeDtypeStruct((n_chunks, N_PAD, B), jnp.float32),
    )(xp, w1, degp)


def _mid_body(s1_ref, y1_ref, degp_ref, b1_ref, w2_ref, h_ref, y2_ref):
    dinv = _dinv_block(degp_ref)
    acc = jnp.zeros((256, B), jnp.float32)
    for cch in range(s1_ref.shape[0]):
        hc = jax.nn.relu(dinv * (s1_ref[cch] + y1_ref[cch])
                         + b1_ref[0, cch * B:(cch + 1) * B])
        h_ref[:, cch * B:(cch + 1) * B] = hc
        acc = acc + jnp.dot(hc, w2_ref[cch],
                            preferred_element_type=jnp.float32)
    y2_ref[0] = acc * dinv


def _layer_mid(s1, y1c, degp, b1, w2r, n_out_chunks):
    n_chunks = s1.shape[0]
    hid = n_chunks * B
    grid = (N_PAD // 256, n_out_chunks)
    return pl.pallas_call(
        _mid_body,
        grid=grid,
        in_specs=[
            pl.BlockSpec((n_chunks, 256, B), lambda i, j: (0, i, 0)),
            pl.BlockSpec((n_chunks, 256, B), lambda i, j: (0, i, 0)),
            pl.BlockSpec((NC, 256, B), lambda i, j: (0, i, 0)),
            pl.BlockSpec((1, hid), lambda i, j: (0, 0)),
            pl.BlockSpec((n_chunks, B, B), lambda i, j: (0, 0, j)),
        ],
        out_specs=[
            pl.BlockSpec((256, hid), lambda i, j: (i, 0)),
            pl.BlockSpec((1, 256, B), lambda i, j: (j, i, 0)),
        ],
        out_shape=[
            jax.ShapeDtypeStruct((N_PAD, hid), jnp.float32),
            jax.ShapeDtypeStruct((n_out_chunks, N_PAD, B), jnp.float32),
        ],
    )(s1, y1c, degp, b1, w2r)


def _final_body(s2_ref, y2_ref, degp_ref, b2_ref, out_ref):
    dinv = _dinv_block(degp_ref)
    for cch in range(s2_ref.shape[0]):
        out_ref[:, cch * B:(cch + 1) * B] = (
            dinv * (s2_ref[cch] + y2_ref[cch])
            + b2_ref[0, cch * B:(cch + 1) * B])


def _layer_final(s2, y2c, degp, b2):
    n_chunks = s2.shape[0]
    odim = n_chunks * B
    return pl.pallas_call(
        _final_body,
        grid=(N_PAD // 256,),
        in_specs=[
            pl.BlockSpec((n_chunks, 256, B), lambda i: (0, i, 0)),
            pl.BlockSpec((n_chunks, 256, B), lambda i: (0, i, 0)),
            pl.BlockSpec((NC, 256, B), lambda i: (0, i, 0)),
            pl.BlockSpec((1, odim), lambda i: (0, 0)),
        ],
        out_specs=pl.BlockSpec((256, odim), lambda i: (i, 0)),
        out_shape=jax.ShapeDtypeStruct((N_PAD, odim), jnp.float32),
    )(s2, y2c, degp, b2)


def kernel(x, edge_index, W1, b1, W2, b2):
    n, in_dim = x.shape
    hid = W1.shape[1]
    odim = W2.shape[1]
    e = edge_index.shape[1]

    epb_tile = -(-e // (NS * B))
    if epb_tile % 2:
        epb_tile += 1
    e_pad = NS * epb_tile * B
    pad = jnp.full((e_pad - e,), n, jnp.int32)
    src = jnp.concatenate([edge_index[0].astype(jnp.int32), pad])
    dst = jnp.concatenate([edge_index[1].astype(jnp.int32), pad])
    src3 = src.reshape(NS, epb_tile, B)
    dst3 = dst.reshape(NS, epb_tile, B)
    dst4 = dst.reshape(NC, NS, epb_tile // 2, B)

    xp = jnp.zeros((N_PAD, in_dim), x.dtype).at[:n].set(x)
    b1r = b1.reshape(1, hid)
    b2r = b2.reshape(1, odim)
    w2r = W2.reshape(-1, B, odim)

    degp = _degree_kernel(dst4)
    n_chunks1 = hid // B
    y1c = _layer1_matmul(xp, W1, degp, n_chunks1)
    s1 = _aggregate([y1c[i] for i in range(n_chunks1)], src3, dst3)
    n_chunks2 = odim // B
    h_pad, y2c = _layer_mid(s1, y1c, degp, b1r, w2r, n_chunks2)
    s2 = _aggregate([y2c[i] for i in range(n_chunks2)], src3, dst3)
    out_pad = _layer_final(s2, y2c, degp, b2r)

    return (out_pad[:n], h_pad[:n])
